# baseline (device time: 295224 ns/iter reference)
import jax

jax.config.update("jax_compilation_cache_dir", "/tmp/jax_comp_cache")
jax.config.update("jax_persistent_cache_min_compile_time_secs", 1.0)

import jax.numpy as jnp
from jax import lax
from jax.experimental import pallas as pl
from jax.experimental.pallas import tpu as pltpu

N_DEV = 4
M_BLK = 2048
K_BLK = 2048
M_Q = M_BLK // 4
N = 4096
NQ = 512
N_STEPS = N // NQ


def kernel(x, w_mat):
    x_bf = x.astype(jnp.bfloat16)

    def _dot(a, b):
        return lax.dot_general(a, b, (((1,), (0,)), ((), ())),
                               preferred_element_type=jnp.float32)

    def body(x_ref, w_ref, out_ref, a_recv, wq, local_sem, w_sems,
             send_sems, recv_sems, diag_sems):
        my = lax.axis_index("i")
        right = (my + 1) % N_DEV
        left = (my + 3) % N_DEV
        diag = (my + 2) % N_DEV

        def send_to(dst, sem_idx):
            return pltpu.make_async_remote_copy(
                src_ref=x_ref.at[pl.ds(dst * M_BLK, M_BLK), :],
                dst_ref=a_recv.at[my],
                send_sem=send_sems.at[sem_idx],
                recv_sem=recv_sems.at[my],
                device_id=(dst,),
                device_id_type=pltpu.DeviceIdType.MESH,
            )

        rd_r = send_to(right, 0)
        rd_l = send_to(left, 1)
        rd_r.start()
        rd_l.start()

        cp_local = pltpu.make_async_copy(
            x_ref.at[pl.ds(my * M_BLK, M_BLK), :], a_recv.at[my], local_sem)
        cp_local.start()

        def fold(s, m0, nm, first):
            def w_load(q, slot):
                return pltpu.make_async_copy(
                    w_ref.at[pl.ds(s * K_BLK, K_BLK), pl.ds(q * NQ, NQ)],
                    wq.at[slot], w_sems.at[slot])

            w_load(0, 0).start()

            def q_step(q, carry):
                slot = lax.rem(q, 2)

                @pl.when(q + 1 < N_STEPS)
                def _():
                    w_load(q + 1, 1 - slot).start()

                pltpu.make_async_copy(wq.at[0], wq.at[0], w_sems.at[slot]).wait()
                c = _dot(a_recv[s, m0:m0 + nm, :], wq[slot])
                qs = pl.ds(q * NQ, NQ)
                ms = pl.ds(m0, nm)
                if first:
                    out_ref[ms, qs] = c.astype(jnp.bfloat16)
                else:
                    out_ref[ms, qs] = (out_ref[ms, qs].astype(jnp.float32)
                                       + c).astype(jnp.bfloat16)
                return carry

            lax.fori_loop(0, N_STEPS, q_step, 0)

        def wait_recv_from(s):
            pltpu.make_async_remote_copy(
                src_ref=a_recv.at[s], dst_ref=a_recv.at[s],
                send_sem=send_sems.at[0], recv_sem=recv_sems.at[s],
                device_id=(my,), device_id_type=pltpu.DeviceIdType.MESH,
            ).wait_recv()

        cp_local.wait()
        fold(my, 0, M_BLK, first=True)

        rd_r.wait_send()
        rd_l.wait_send()
        diag_chunks = []
        for h in range(4):
            rd = pltpu.make_async_remote_copy(
                src_ref=x_ref.at[pl.ds(diag * M_BLK + h * M_Q, M_Q), :],
                dst_ref=a_recv.at[my, pl.ds(h * M_Q, M_Q), :],
                send_sem=send_sems.at[2 + h],
                recv_sem=diag_sems.at[h],
                device_id=(diag,),
                device_id_type=pltpu.DeviceIdType.MESH,
            )
            rd.start()
            diag_chunks.append(rd)

        for s in (left, right):
            wait_recv_from(s)
            fold(s, 0, M_BLK, first=False)
        for h in range(4):
            ref = a_recv.at[diag, pl.ds(h * M_Q, M_Q), :]
            pltpu.make_async_remote_copy(
                src_ref=ref, dst_ref=ref,
                send_sem=send_sems.at[0], recv_sem=diag_sems.at[h],
                device_id=(my,), device_id_type=pltpu.DeviceIdType.MESH,
            ).wait_recv()
            fold(diag, h * M_Q, M_Q, first=False)

        for rd in diag_chunks:
            rd.wait_send()

    return pl.pallas_call(
        body,
        out_shape=jax.ShapeDtypeStruct((M_BLK, N), jnp.bfloat16),
        in_specs=[
            pl.BlockSpec(memory_space=pl.ANY),
            pl.BlockSpec(memory_space=pl.ANY),
        ],
        out_specs=pl.BlockSpec(memory_space=pltpu.VMEM),
        scratch_shapes=[
            pltpu.VMEM((N_DEV, M_BLK, K_BLK), jnp.bfloat16),
            pltpu.VMEM((2, K_BLK, NQ), jnp.float32),
            pltpu.SemaphoreType.DMA,
            pltpu.SemaphoreType.DMA((2,)),
            pltpu.SemaphoreType.DMA((6,)),
            pltpu.SemaphoreType.DMA((N_DEV,)),
            pltpu.SemaphoreType.DMA((4,)),
        ],
        compiler_params=pltpu.CompilerParams(
            vmem_limit_bytes=63 * 1024 * 1024,
        ),
    )(x_bf, w_mat)


# device time: 274979 ns/iter; 1.0736x vs baseline; 1.0736x over previous
import jax

jax.config.update("jax_compilation_cache_dir", "/tmp/jax_comp_cache")
jax.config.update("jax_persistent_cache_min_compile_time_secs", 1.0)

import jax.numpy as jnp
from jax import lax
from jax.experimental import pallas as pl
from jax.experimental.pallas import tpu as pltpu

N_DEV = 4
M_BLK = 2048
K_BLK = 2048
M_HALF = M_BLK // 2
N = 4096
NQ = 512
N_STEPS = N // NQ


def kernel(x, w_mat):
    x_bf = x.astype(jnp.bfloat16)

    def _dot(a, b):
        return lax.dot_general(a, b, (((1,), (0,)), ((), ())),
                               preferred_element_type=jnp.float32)

    def body(x_ref, w_ref, out_ref, a_recv, wq, local_sem, w_sems,
             send_sems, recv_sems, diag_sems):
        my = lax.axis_index("i")
        right = (my + 1) % N_DEV
        left = (my + 3) % N_DEV
        diag = (my + 2) % N_DEV

        def send_to(dst, sem_idx):
            return pltpu.make_async_remote_copy(
                src_ref=x_ref.at[pl.ds(dst * M_BLK, M_BLK), :],
                dst_ref=a_recv.at[my],
                send_sem=send_sems.at[sem_idx],
                recv_sem=recv_sems.at[my],
                device_id=(dst,),
                device_id_type=pltpu.DeviceIdType.MESH,
            )

        rd_r = send_to(right, 0)
        rd_l = send_to(left, 1)
        rd_r.start()
        rd_l.start()

        cp_local = pltpu.make_async_copy(
            x_ref.at[pl.ds(my * M_BLK, M_BLK), :], a_recv.at[my], local_sem)
        cp_local.start()

        def fold(s, m0, nm, first):
            def w_load(q, slot):
                return pltpu.make_async_copy(
                    w_ref.at[pl.ds(s * K_BLK, K_BLK), pl.ds(q * NQ, NQ)],
                    wq.at[slot], w_sems.at[slot])

            w_load(0, 0).start()

            def q_step(q, carry):
                slot = lax.rem(q, 2)

                @pl.when(q + 1 < N_STEPS)
                def _():
                    w_load(q + 1, 1 - slot).start()

                pltpu.make_async_copy(wq.at[0], wq.at[0], w_sems.at[slot]).wait()
                c = _dot(a_recv[s, m0:m0 + nm, :], wq[slot])
                qs = pl.ds(q * NQ, NQ)
                ms = pl.ds(m0, nm)
                if first:
                    out_ref[ms, qs] = c.astype(jnp.bfloat16)
                else:
                    out_ref[ms, qs] = (out_ref[ms, qs].astype(jnp.float32)
                                       + c).astype(jnp.bfloat16)
                return carry

            lax.fori_loop(0, N_STEPS, q_step, 0)

        def wait_recv_from(s):
            pltpu.make_async_remote_copy(
                src_ref=a_recv.at[s], dst_ref=a_recv.at[s],
                send_sem=send_sems.at[0], recv_sem=recv_sems.at[s],
                device_id=(my,), device_id_type=pltpu.DeviceIdType.MESH,
            ).wait_recv()

        cp_local.wait()
        fold(my, 0, M_BLK, first=True)

        rd_r.wait_send()
        rd_l.wait_send()
        diag_halves = []
        for h in range(2):
            rd = pltpu.make_async_remote_copy(
                src_ref=x_ref.at[pl.ds(diag * M_BLK + h * M_HALF, M_HALF), :],
                dst_ref=a_recv.at[my, pl.ds(h * M_HALF, M_HALF), :],
                send_sem=send_sems.at[2 + h],
                recv_sem=diag_sems.at[h],
                device_id=(diag,),
                device_id_type=pltpu.DeviceIdType.MESH,
            )
            rd.start()
            diag_halves.append(rd)

        for s in (left, right):
            wait_recv_from(s)
            fold(s, 0, M_BLK, first=False)
        for h in range(2):
            ref = a_recv.at[diag, pl.ds(h * M_HALF, M_HALF), :]
            pltpu.make_async_remote_copy(
                src_ref=ref, dst_ref=ref,
                send_sem=send_sems.at[0], recv_sem=diag_sems.at[h],
                device_id=(my,), device_id_type=pltpu.DeviceIdType.MESH,
            ).wait_recv()
            fold(diag, h * M_HALF, M_HALF, first=False)

        for rd in diag_halves:
            rd.wait_send()

    return pl.pallas_call(
        body,
        out_shape=jax.ShapeDtypeStruct((M_BLK, N), jnp.bfloat16),
        in_specs=[
            pl.BlockSpec(memory_space=pl.ANY),
            pl.BlockSpec(memory_space=pl.ANY),
        ],
        out_specs=pl.BlockSpec(memory_space=pltpu.VMEM),
        scratch_shapes=[
            pltpu.VMEM((N_DEV, M_BLK, K_BLK), jnp.bfloat16),
            pltpu.VMEM((2, K_BLK, NQ), jnp.float32),
            pltpu.SemaphoreType.DMA,
            pltpu.SemaphoreType.DMA((2,)),
            pltpu.SemaphoreType.DMA((4,)),
            pltpu.SemaphoreType.DMA((N_DEV,)),
            pltpu.SemaphoreType.DMA((2,)),
        ],
        compiler_params=pltpu.CompilerParams(
            vmem_limit_bytes=63 * 1024 * 1024,
        ),
    )(x_bf, w_mat)


# device time: 274872 ns/iter; 1.0740x vs baseline; 1.0004x over previous
import jax

jax.config.update("jax_compilation_cache_dir", "/tmp/jax_comp_cache")
jax.config.update("jax_persistent_cache_min_compile_time_secs", 1.0)

import jax.numpy as jnp
from jax import lax
from jax.experimental import pallas as pl
from jax.experimental.pallas import tpu as pltpu

N_DEV = 4
M_BLK = 2048
K_BLK = 2048
M_HALF = M_BLK // 2
N = 4096
NQ = 512
N_STEPS = N // NQ


def kernel(x, w_mat):
    x_bf = x.astype(jnp.bfloat16)

    def _dot(a, b):
        return lax.dot_general(a, b, (((1,), (0,)), ((), ())),
                               preferred_element_type=jnp.float32)

    def body(x_ref, w_ref, out_ref, a_recv, wq, local_sem, w_sems,
             send_sems, recv_sems, diag_sems):
        my = lax.axis_index("i")
        right = (my + 1) % N_DEV
        left = (my + 3) % N_DEV
        diag = (my + 2) % N_DEV

        def send_to(dst, sem_idx):
            return pltpu.make_async_remote_copy(
                src_ref=x_ref.at[pl.ds(dst * M_BLK, M_BLK), :],
                dst_ref=a_recv.at[my],
                send_sem=send_sems.at[sem_idx],
                recv_sem=recv_sems.at[my],
                device_id=(dst,),
                device_id_type=pltpu.DeviceIdType.MESH,
            )

        rd_r = send_to(right, 0)
        rd_l = send_to(left, 1)
        rd_r.start()
        rd_l.start()

        cp_local = pltpu.make_async_copy(
            x_ref.at[pl.ds(my * M_BLK, M_BLK), :], a_recv.at[my], local_sem)
        cp_local.start()

        def fold(s, m0, nm, first):
            def w_load(q, slot):
                return pltpu.make_async_copy(
                    w_ref.at[pl.ds(s * K_BLK, K_BLK), pl.ds(q * NQ, NQ)],
                    wq.at[slot], w_sems.at[slot])

            w_load(0, 0).start()

            def q_step(q, carry):
                slot = lax.rem(q, 2)

                @pl.when(q + 1 < N_STEPS)
                def _():
                    w_load(q + 1, 1 - slot).start()

                pltpu.make_async_copy(wq.at[0], wq.at[0], w_sems.at[slot]).wait()
                c = _dot(a_recv[s, m0:m0 + nm, :], wq[slot])
                qs = pl.ds(q * NQ, NQ)
                ms = pl.ds(m0, nm)
                if first:
                    out_ref[ms, qs] = c.astype(jnp.bfloat16)
                else:
                    out_ref[ms, qs] = (out_ref[ms, qs].astype(jnp.float32)
                                       + c).astype(jnp.bfloat16)
                return carry

            lax.fori_loop(0, N_STEPS, q_step, 0)

        def wait_recv_from(s):
            pltpu.make_async_remote_copy(
                src_ref=a_recv.at[s], dst_ref=a_recv.at[s],
                send_sem=send_sems.at[0], recv_sem=recv_sems.at[s],
                device_id=(my,), device_id_type=pltpu.DeviceIdType.MESH,
            ).wait_recv()

        cp_local.wait()
        fold(my, 0, M_BLK, first=True)

        wait_recv_from(left)
        diag_halves = []
        for h in range(2):
            rd = pltpu.make_async_remote_copy(
                src_ref=x_ref.at[pl.ds(diag * M_BLK + h * M_HALF, M_HALF), :],
                dst_ref=a_recv.at[my, pl.ds(h * M_HALF, M_HALF), :],
                send_sem=send_sems.at[2 + h],
                recv_sem=diag_sems.at[h],
                device_id=(diag,),
                device_id_type=pltpu.DeviceIdType.MESH,
            )
            rd.start()
            diag_halves.append(rd)

        fold(left, 0, M_BLK, first=False)
        wait_recv_from(right)
        fold(right, 0, M_BLK, first=False)
        for h in range(2):
            ref = a_recv.at[diag, pl.ds(h * M_HALF, M_HALF), :]
            pltpu.make_async_remote_copy(
                src_ref=ref, dst_ref=ref,
                send_sem=send_sems.at[0], recv_sem=diag_sems.at[h],
                device_id=(my,), device_id_type=pltpu.DeviceIdType.MESH,
            ).wait_recv()
            fold(diag, h * M_HALF, M_HALF, first=False)

        rd_r.wait_send()
        rd_l.wait_send()
        for rd in diag_halves:
            rd.wait_send()

    return pl.pallas_call(
        body,
        out_shape=jax.ShapeDtypeStruct((M_BLK, N), jnp.bfloat16),
        in_specs=[
            pl.BlockSpec(memory_space=pl.ANY),
            pl.BlockSpec(memory_space=pl.ANY),
        ],
        out_specs=pl.BlockSpec(memory_space=pltpu.VMEM),
        scratch_shapes=[
            pltpu.VMEM((N_DEV, M_BLK, K_BLK), jnp.bfloat16),
            pltpu.VMEM((2, K_BLK, NQ), jnp.float32),
            pltpu.SemaphoreType.DMA,
            pltpu.SemaphoreType.DMA((2,)),
            pltpu.SemaphoreType.DMA((4,)),
            pltpu.SemaphoreType.DMA((N_DEV,)),
            pltpu.SemaphoreType.DMA((2,)),
        ],
        compiler_params=pltpu.CompilerParams(
            vmem_limit_bytes=63 * 1024 * 1024,
        ),
    )(x_bf, w_mat)
